# baseline (speedup 1.0000x reference)
import functools

import jax
import jax.numpy as jnp
from jax import lax
from jax.experimental import pallas as pl
from jax.experimental.pallas import tpu as pltpu
from jax.experimental.pallas import tpu_sc as plsc

N = 10000
NP = 10240
D = 128
E = 320000
NC = 2
NS = 16
NW = NC * NS
CHUNK = 128
NCH = 80
EPT = NCH * CHUNK
EPAD = NW * EPT
GRP = 4
NGRP = NCH // GRP
ROWS_T = NP // NS
DUMP = N

_mesh = plsc.VectorSubcoreMesh(core_axis_name="c", subcore_axis_name="s")

_f32 = jnp.float32


def _zeros16():
    return jnp.zeros((16,), _f32)


def _ones16():
    return jnp.ones((16,), _f32)


@functools.partial(
    pl.kernel,
    out_type=jax.ShapeDtypeStruct((NC, 1, NP), _f32),
    mesh=_mesh,
    scratch_types=[
        pltpu.VMEM((NCH, 1, CHUNK), jnp.int32),
        pltpu.VMEM((CHUNK,), jnp.int32),
        pltpu.VMEM((CHUNK,), jnp.int32),
        pltpu.VMEM((CHUNK,), jnp.int32),
        pltpu.VMEM((CHUNK,), jnp.int32),
        pltpu.VMEM((CHUNK,), _f32),
        pltpu.VMEM((ROWS_T,), _f32),
        pltpu.VMEM_SHARED((NP,), _f32),
        pltpu.SemaphoreType.DMA,
    ],
)
def _deg_kernel(dst_hbm, out_hbm, idxall, ib0, ib1, ib2, ib3, ones_v, zb,
                deg_sh, sem):
    c = lax.axis_index("c")
    s = lax.axis_index("s")
    w = c * NS + s
    ibs = (ib0, ib1, ib2, ib3)

    for j in range(CHUNK // 16):
        ones_v[pl.ds(j * 16, 16)] = _ones16()

    @pl.loop(0, ROWS_T // 16)
    def _z(i):
        zb[pl.ds(i * 16, 16)] = _zeros16()

    pltpu.sync_copy(zb, deg_sh.at[pl.ds(s * ROWS_T, ROWS_T)])
    plsc.subcore_barrier()

    pltpu.sync_copy(dst_hbm.at[pl.ds(w * NCH, NCH)], idxall)

    @pl.loop(0, NGRP)
    def _grp(g):
        base = g * GRP
        descs = []
        for k in range(GRP):
            for j in range(CHUNK // 16):
                ibs[k][pl.ds(j * 16, 16)] = idxall[base + k, 0,
                                                   pl.ds(j * 16, 16)]
            d = pltpu.make_async_copy(ones_v, deg_sh.at[ibs[k]], sem)
            d.start(add=True)
            descs.append(d)
        for d in descs:
            d.wait()

    plsc.subcore_barrier()
    pltpu.sync_copy(deg_sh.at[pl.ds(s * ROWS_T, ROWS_T)],
                    out_hbm.at[c, 0, pl.ds(s * ROWS_T, ROWS_T)])


@functools.partial(
    pl.kernel,
    out_type=jax.ShapeDtypeStruct((NC, NP, D), _f32),
    mesh=_mesh,
    scratch_types=[
        pltpu.VMEM((NCH, 1, CHUNK), jnp.int32),
        pltpu.VMEM((NCH, 1, CHUNK), jnp.int32),
        pltpu.VMEM((CHUNK,), jnp.int32),
        pltpu.VMEM((CHUNK,), jnp.int32),
        pltpu.VMEM((CHUNK, D), _f32),
        pltpu.VMEM_SHARED((NP, D), _f32),
        pltpu.SemaphoreType.DMA,
        pltpu.SemaphoreType.DMA,
    ],
)
def _hop_kernel(tab_hbm, src_hbm, dst_hbm, out_hbm, sidx, didx,
                ibs, ibd, rb_g, acc, gsem, ssem):
    c = lax.axis_index("c")
    s = lax.axis_index("s")
    w = c * NS + s

    @pl.loop(0, CHUNK)
    def _z(i):
        for j in range(D // 16):
            rb_g[i, pl.ds(j * 16, 16)] = _zeros16()

    for r in range(ROWS_T // CHUNK):
        pltpu.sync_copy(rb_g, acc.at[pl.ds(s * ROWS_T + r * CHUNK, CHUNK)])
    plsc.subcore_barrier()

    pltpu.sync_copy(src_hbm.at[pl.ds(w * NCH, NCH)], sidx)
    pltpu.sync_copy(dst_hbm.at[pl.ds(w * NCH, NCH)], didx)

    @pl.loop(0, NCH)
    def _chunk(ch):
        for j in range(CHUNK // 16):
            ibs[pl.ds(j * 16, 16)] = sidx[ch, 0, pl.ds(j * 16, 16)]
            ibd[pl.ds(j * 16, 16)] = didx[ch, 0, pl.ds(j * 16, 16)]
        g = pltpu.make_async_copy(tab_hbm.at[ibs], rb_g, gsem)
        g.start()
        g.wait()
        sct = pltpu.make_async_copy(rb_g, acc.at[ibd], ssem)
        sct.start(add=True)
        sct.wait()

    plsc.subcore_barrier()
    pltpu.sync_copy(acc.at[pl.ds(s * ROWS_T, ROWS_T)],
                    out_hbm.at[c, pl.ds(s * ROWS_T, ROWS_T)])


def _mm_scales_body(x_ref, w_ref, d_ref, o_ref, sc_ref):
    deg = jnp.maximum(d_ref[0, 0, :] + d_ref[1, 0, :], 1.0)
    sc_ref[0, :] = lax.rsqrt(deg)
    sc_ref[1, :] = 1.0 / deg
    o_ref[0] = jnp.dot(x_ref[...], w_ref[...], preferred_element_type=_f32)
    o_ref[1] = jnp.zeros((NP, D), _f32)


def _comb_scale_body(p_ref, s_ref, o_ref):
    o_ref[...] = (p_ref[0] + p_ref[1]) * s_ref[...][:, None]


def _final_body(q_ref, s_ref, b_ref, o_ref):
    o_ref[...] = (q_ref[0] + q_ref[1]) * s_ref[0, :][:, None] \
        + b_ref[...][None, :]


_mm_scales = pl.pallas_call(
    _mm_scales_body,
    out_shape=(jax.ShapeDtypeStruct((NC, NP, D), _f32),
               jax.ShapeDtypeStruct((2, NP), _f32)))
_comb_scale = pl.pallas_call(
    _comb_scale_body, out_shape=jax.ShapeDtypeStruct((NP, D), _f32))
_final = pl.pallas_call(
    _final_body, out_shape=jax.ShapeDtypeStruct((NP, D), _f32))


def kernel(feat, edge_index, W, b):
    src = edge_index[0]
    dst = edge_index[1]
    pad = EPAD - E
    pad_row = DUMP + (jnp.arange(pad, dtype=jnp.int32) % (NP - N))
    srcp = jnp.concatenate([src, pad_row]).reshape(NW * NCH, 1, CHUNK)
    dstp = jnp.concatenate([dst, pad_row]).reshape(NW * NCH, 1, CHUNK)
    featp = jnp.pad(feat, ((0, NP - N), (0, 0)))

    deg01 = _deg_kernel(dstp)
    p, scales = _mm_scales(featp, W, deg01)

    def _body(carry, scale_i):
        y = _comb_scale(carry, scale_i)
        return _hop_kernel(y, srcp, dstp), None

    q, _ = lax.scan(_body, p, scales)
    outp = _final(q, scales, b)
    return outp[:N]


# speedup vs baseline: 1.0318x; 1.0318x over previous
import functools

import jax
import jax.numpy as jnp
from jax import lax
from jax.experimental import pallas as pl
from jax.experimental.pallas import tpu as pltpu
from jax.experimental.pallas import tpu_sc as plsc

N = 10000
NP = 10240
D = 128
E = 320000
NC = 2
NS = 16
NW = NC * NS
CHUNK = 128
NCH = 80
EPT = NCH * CHUNK
EPAD = NW * EPT
GRP = 4
NGRP = NCH // GRP
ROWS_T = NP // NS
DUMP = N

_mesh = plsc.VectorSubcoreMesh(core_axis_name="c", subcore_axis_name="s")

_f32 = jnp.float32


def _zeros16():
    return jnp.zeros((16,), _f32)


def _ones16():
    return jnp.ones((16,), _f32)


@functools.partial(
    pl.kernel,
    out_type=jax.ShapeDtypeStruct((NC, 1, NP), _f32),
    mesh=_mesh,
    scratch_types=[
        pltpu.VMEM((NCH, 1, CHUNK), jnp.int32),
        pltpu.VMEM((CHUNK,), jnp.int32),
        pltpu.VMEM((CHUNK,), jnp.int32),
        pltpu.VMEM((CHUNK,), jnp.int32),
        pltpu.VMEM((CHUNK,), jnp.int32),
        pltpu.VMEM((CHUNK,), _f32),
        pltpu.VMEM((ROWS_T,), _f32),
        pltpu.VMEM_SHARED((NP,), _f32),
        pltpu.SemaphoreType.DMA,
    ],
)
def _deg_kernel(dst_hbm, out_hbm, idxall, ib0, ib1, ib2, ib3, ones_v, zb,
                deg_sh, sem):
    c = lax.axis_index("c")
    s = lax.axis_index("s")
    w = c * NS + s
    ibs = (ib0, ib1, ib2, ib3)

    for j in range(CHUNK // 16):
        ones_v[pl.ds(j * 16, 16)] = _ones16()

    @pl.loop(0, ROWS_T // 16)
    def _z(i):
        zb[pl.ds(i * 16, 16)] = _zeros16()

    pltpu.sync_copy(zb, deg_sh.at[pl.ds(s * ROWS_T, ROWS_T)])
    plsc.subcore_barrier()

    pltpu.sync_copy(dst_hbm.at[pl.ds(w * NCH, NCH)], idxall)

    @pl.loop(0, NGRP)
    def _grp(g):
        base = g * GRP
        descs = []
        for k in range(GRP):
            for j in range(CHUNK // 16):
                ibs[k][pl.ds(j * 16, 16)] = idxall[base + k, 0,
                                                   pl.ds(j * 16, 16)]
            d = pltpu.make_async_copy(ones_v, deg_sh.at[ibs[k]], sem)
            d.start(add=True)
            descs.append(d)
        for d in descs:
            d.wait()

    plsc.subcore_barrier()
    pltpu.sync_copy(deg_sh.at[pl.ds(s * ROWS_T, ROWS_T)],
                    out_hbm.at[c, 0, pl.ds(s * ROWS_T, ROWS_T)])


@functools.partial(
    pl.kernel,
    out_type=jax.ShapeDtypeStruct((NC, NP, D), _f32),
    mesh=_mesh,
    scratch_types=[
        pltpu.VMEM((NCH, 1, CHUNK), jnp.int32),
        pltpu.VMEM((NCH, 1, CHUNK), jnp.int32),
        pltpu.VMEM((CHUNK,), jnp.int32),
        pltpu.VMEM((CHUNK,), jnp.int32),
        pltpu.VMEM((CHUNK, D), _f32),
        pltpu.VMEM_SHARED((NP, D), _f32),
        pltpu.SemaphoreType.DMA,
        pltpu.SemaphoreType.DMA,
    ],
)
def _hop_kernel(tab_hbm, src_hbm, dst_hbm, out_hbm, sidx, didx,
                ibs, ibd, rb_g, acc, gsem, ssem):
    c = lax.axis_index("c")
    s = lax.axis_index("s")
    w = c * NS + s

    @pl.loop(0, CHUNK)
    def _z(i):
        for j in range(D // 16):
            rb_g[i, pl.ds(j * 16, 16)] = _zeros16()

    for r in range(ROWS_T // CHUNK):
        pltpu.sync_copy(rb_g, acc.at[pl.ds(s * ROWS_T + r * CHUNK, CHUNK)])
    plsc.subcore_barrier()

    pltpu.sync_copy(src_hbm.at[pl.ds(w * NCH, NCH)], sidx)
    pltpu.sync_copy(dst_hbm.at[pl.ds(w * NCH, NCH)], didx)

    @pl.loop(0, NCH)
    def _chunk(ch):
        for j in range(CHUNK // 16):
            ibs[pl.ds(j * 16, 16)] = sidx[ch, 0, pl.ds(j * 16, 16)]
            ibd[pl.ds(j * 16, 16)] = didx[ch, 0, pl.ds(j * 16, 16)]
        g = pltpu.make_async_copy(tab_hbm.at[ibs], rb_g, gsem)
        g.start()
        g.wait()
        sct = pltpu.make_async_copy(rb_g, acc.at[ibd], ssem)
        sct.start(add=True)
        sct.wait()

    plsc.subcore_barrier()
    pltpu.sync_copy(acc.at[pl.ds(s * ROWS_T, ROWS_T)],
                    out_hbm.at[c, pl.ds(s * ROWS_T, ROWS_T)])


def _mm_scales_body(x_ref, w_ref, d_ref, y_ref, sc_ref):
    deg = jnp.maximum(d_ref[0, 0, :] + d_ref[1, 0, :], 1.0)
    nrm = lax.rsqrt(deg)
    sc_ref[0, :] = 1.0 / deg
    sc_ref[1, :] = nrm
    y = jnp.dot(x_ref[...], w_ref[...], preferred_element_type=_f32)
    y_ref[...] = y * nrm[:, None]


def _comb_scale_body(p_ref, s_ref, b_ref, o_ref):
    o_ref[...] = (p_ref[0] + p_ref[1]) * s_ref[...][:, None] \
        + b_ref[...][None, :]


_mm_scales = pl.pallas_call(
    _mm_scales_body,
    out_shape=(jax.ShapeDtypeStruct((NP, D), _f32),
               jax.ShapeDtypeStruct((2, NP), _f32)))
_comb_scale = pl.pallas_call(
    _comb_scale_body, out_shape=jax.ShapeDtypeStruct((NP, D), _f32))


def kernel(feat, edge_index, W, b):
    src = edge_index[0]
    dst = edge_index[1]
    pad = EPAD - E
    pad_row = DUMP + (jnp.arange(pad, dtype=jnp.int32) % (NP - N))
    srcp = jnp.concatenate([src, pad_row]).reshape(NW * NCH, 1, CHUNK)
    dstp = jnp.concatenate([dst, pad_row]).reshape(NW * NCH, 1, CHUNK)
    featp = jnp.pad(feat, ((0, NP - N), (0, 0)))

    deg01 = _deg_kernel(dstp)
    y0, scales = _mm_scales(featp, W, deg01)
    bias_xs = jnp.stack([jnp.zeros((D,), _f32), b])

    def _body(y, xs):
        scale_i, b_i = xs
        p = _hop_kernel(y, srcp, dstp)
        return _comb_scale(p, scale_i, b_i), None

    y2, _ = lax.scan(_body, y0, (scales, bias_xs))
    return y2[:N]
